# baseline (speedup 1.0000x reference)
import functools

import jax
import jax.numpy as jnp
from jax import lax
from jax.experimental import pallas as pl
from jax.experimental.pallas import tpu as pltpu
from jax.experimental.pallas import tpu_sc as plsc

N = 10000
E = 640000
SEQ = 12
GD = 16
NPAD = 10240
BM = 400
BK = 2048

NC, NS = 2, 16
NW = NC * NS
EC = 128
E2 = E + N
E2P = -(-E2 // (NW * EC * 2)) * (NW * EC * 2)
SPAN = E2P // NW
CH = SPAN // EC
NA = 10112
RPT = NA // NS


def _adp_body(q_ref, k_ref, v_ref, out_ref):
    q = q_ref[...]

    def body(i, acc):
        kc = k_ref[:, pl.ds(i * BK, BK)]
        logits = jnp.dot(q, kc, preferred_element_type=jnp.float32)
        p = jnp.exp(jnp.maximum(logits, 0.0))
        return acc + jnp.dot(p, v_ref[pl.ds(i * BK, BK), :],
                             preferred_element_type=jnp.float32)

    acc = jax.lax.fori_loop(
        0, NPAD // BK, body, jnp.zeros((BM, GD + 1), jnp.float32))
    out_ref[...] = acc[:, :GD] / acc[:, GD:GD + 1]


@jax.jit
def _adp_all(q, k_aug, v):
    ones = jnp.ones((N, 1), jnp.float32)
    v_aug = jnp.concatenate([v, ones], axis=1)
    v_aug = jnp.pad(v_aug, ((0, NPAD - N), (0, 0)))
    return pl.pallas_call(
        _adp_body,
        grid=(N // BM,),
        in_specs=[
            pl.BlockSpec((BM, q.shape[1]), lambda i: (i, 0)),
            pl.BlockSpec((q.shape[1], NPAD), lambda i: (0, 0)),
            pl.BlockSpec((NPAD, GD + 1), lambda i: (0, 0)),
        ],
        out_specs=pl.BlockSpec((BM, GD), lambda i: (i, 0)),
        out_shape=jax.ShapeDtypeStruct((N, GD), jnp.float32),
    )(q, k_aug, v_aug)


def _gat_body(heads, RW,
              h_hbm, as_hbm, ad_hbm, src_hbm, dst_hbm, out_hbm,
              as_v, ad_v, srcb, dstb, rows_ab, buf_ab, accum,
              sem_ra, sem_rb, sem_sa, sem_sb,
              sem_i0, sem_i1, sem_i2, sem_i3):
    c = lax.axis_index("c")
    s = lax.axis_index("s")
    wid = c * NS + s
    pltpu.sync_copy(as_hbm, as_v)
    pltpu.sync_copy(ad_hbm, ad_v)

    zeros16 = jnp.zeros((16,), jnp.float32)
    zoffs = [j * 16 for j in range(RW // 16)] + [RW - 16]

    def zrow(i, _):
        for ab in range(2):
            for off in zoffs:
                buf_ab[ab, i, pl.ds(off, 16)] = zeros16
        return 0

    lax.fori_loop(0, EC, zrow, 0)

    off, rem = 0, RPT
    while rem > 0:
        k = min(EC, rem)
        pltpu.sync_copy(buf_ab.at[0, pl.ds(0, k)],
                        accum.at[pl.ds(s * RPT + off, k)])
        off += k
        rem -= k
    plsc.subcore_barrier()

    lane = lax.iota(jnp.int32, 16)
    dencol = [jnp.full((16,), heads * GD + hd, jnp.int32)
              for hd in range(heads)]
    col16 = [jnp.full((16,), f, jnp.int32) for f in range(GD)]
    colhf = [[jnp.full((16,), hd * GD + f, jnp.int32) for f in range(GD)]
             for hd in range(heads)]
    sem_r = [sem_ra, sem_rb]
    sem_s = [sem_sa, sem_sb]
    sem_i = [sem_i0, sem_i1, sem_i2, sem_i3]

    def issue_idx(g, sl):
        pltpu.async_copy(src_hbm.at[wid, g], srcb.at[sl], sem_i[sl])
        pltpu.async_copy(dst_hbm.at[wid, g], dstb.at[sl], sem_i[sl])

    def drain_idx(sl):
        pltpu.make_async_copy(src_hbm.at[0, 0], srcb.at[sl],
                              sem_i[sl]).wait()
        pltpu.make_async_copy(dst_hbm.at[0, 0], dstb.at[sl],
                              sem_i[sl]).wait()

    def issue_rows(sl, rs):
        for hd in range(heads):
            pltpu.async_copy(h_hbm.at[hd].at[srcb.at[sl]],
                             rows_ab.at[rs, hd], sem_r[rs])

    def drain_rows(rs):
        pltpu.make_async_copy(h_hbm.at[0].at[srcb.at[0]],
                              rows_ab.at[rs], sem_r[rs]).wait()

    def scatter(sl, rs):
        pltpu.async_copy(buf_ab.at[rs], accum.at[dstb.at[sl]], sem_s[rs],
                         add=True)

    def drain_scatter(rs):
        pltpu.make_async_copy(buf_ab.at[rs], accum.at[dstb.at[0]],
                              sem_s[rs]).wait()

    def compute(sl, rs):
        def grp(i, _):
            sv = srcb[sl, pl.ds(i * 16, 16)]
            dv = dstb[sl, pl.ds(i * 16, 16)]
            ev = i * 16 + lane
            for hd in range(heads):
                av = plsc.load_gather(as_v, [sv + hd * N])
                bv = plsc.load_gather(ad_v, [dv + hd * N])
                t = av + bv
                w = jnp.exp(jnp.where(t > 0, t, 0.2 * t))
                plsc.store_scatter(buf_ab.at[rs], [ev, dencol[hd]], w)
                rh = rows_ab.at[rs, hd]
                for f in range(GD):
                    vals = plsc.load_gather(rh, [ev, col16[f]])
                    plsc.store_scatter(buf_ab.at[rs],
                                       [ev, colhf[hd][f]], vals * w)
            return 0

        lax.fori_loop(0, EC // 16, grp, 0)

    issue_idx(0, 0)
    issue_idx(1, 1)
    drain_idx(0)
    issue_rows(0, 0)

    def body(q, _):
        for c in range(4):
            g = 4 * q + c
            rs = c % 2
            if c < 2:
                @pl.when(q > 0)
                def _():
                    drain_scatter(rs)
            else:
                drain_scatter(rs)

            @pl.when(g + 2 < CH)
            def _():
                issue_idx(g + 2, (c + 2) % 4)

            @pl.when(g + 1 < CH)
            def _():
                drain_idx((c + 1) % 4)
                issue_rows((c + 1) % 4, (c + 1) % 2)

            drain_rows(rs)
            compute(c, rs)
            scatter(c, rs)
        return 0

    lax.fori_loop(0, CH // 4, body, 0)
    drain_scatter(0)
    drain_scatter(1)
    plsc.subcore_barrier()

    off, rem = 0, RPT
    while rem > 0:
        k = min(EC, rem)
        pltpu.sync_copy(accum.at[pl.ds(s * RPT + off, k)],
                        out_hbm.at[c].at[pl.ds(s * RPT + off, k)])
        off += k
        rem -= k


@functools.lru_cache(maxsize=None)
def _gat_call(heads):
    RW = -(-(heads * GD + heads) // 8) * 8
    mesh = plsc.VectorSubcoreMesh(core_axis_name="c", subcore_axis_name="s",
                                  num_cores=NC, num_subcores=NS)
    return pl.kernel(
        functools.partial(_gat_body, heads, RW),
        out_type=jax.ShapeDtypeStruct((NC, NA, RW), jnp.float32),
        mesh=mesh,
        compiler_params=pltpu.CompilerParams(needs_layout_passes=False,
                                             use_tc_tiling_on_sc=False),
        scratch_types=[
            pltpu.VMEM((heads * N,), jnp.float32),
            pltpu.VMEM((heads * N,), jnp.float32),
            pltpu.VMEM((4, EC), jnp.int32),
            pltpu.VMEM((4, EC), jnp.int32),
            pltpu.VMEM((2, heads, EC, GD), jnp.float32),
            pltpu.VMEM((2, EC, RW), jnp.float32),
            pltpu.VMEM_SHARED((NA, RW), jnp.float32),
        ] + [pltpu.SemaphoreType.DMA] * 8,
    )


def _gat(x, src, dst, p):
    heads = p["att_src"].shape[0]
    h = (x @ p["W"]).reshape(N, heads, GD)
    a_s = jnp.sum(h * p["att_src"][None, :, :], axis=-1)
    a_d = jnp.sum(h * p["att_dst"][None, :, :], axis=-1)
    hT = jnp.transpose(h, (1, 0, 2))
    partials = _gat_call(heads)(
        hT, a_s.T.reshape(-1), a_d.T.reshape(-1),
        src.reshape(NW, CH, EC), dst.reshape(NW, CH, EC))
    acc = partials[0] + partials[1]
    num = acc[:N, :heads * GD].reshape(N, heads, GD)
    den = acc[:N, heads * GD:heads * GD + heads]
    out = num / (den[:, :, None] + 1e-16)
    return jnp.mean(out, axis=1) + p["b"]


def _branch(x, ei, bp):
    loop = jnp.arange(N, dtype=jnp.int32)
    src = jnp.pad(jnp.concatenate([ei[0], loop]), (0, E2P - E2))
    dst = jnp.pad(jnp.concatenate([ei[1], loop]), (0, E2P - E2),
                  constant_values=N)
    q = bp["src_emb"]
    k_aug = jnp.pad(bp["tgt_emb"], ((0, 0), (0, NPAD - N)))

    def adp(lp, v):
        return _adp_all(q, k_aug, v @ lp["W"]) + lp["b"]

    g1 = _gat(x, src, dst, bp["g1"])
    s1 = jax.nn.sigmoid(adp(bp["l1"], x))
    origin = x @ bp["origin"]["W"] + bp["origin"]["b"]
    o1 = jnp.tanh(g1) * s1 + origin * (1.0 - s1)

    t1 = jnp.tanh(o1)
    g2 = _gat(t1, src, dst, bp["g2"])
    s2 = jax.nn.sigmoid(adp(bp["l2"], t1))
    o2 = jax.nn.leaky_relu(g2, 0.01) * s2 + o1 * (1.0 - s2)

    r2 = jax.nn.relu(o2)
    g3 = _gat(r2, src, dst, bp["g3"])
    s3 = jax.nn.sigmoid(adp(bp["l3"], r2))
    o3 = jax.nn.relu(g3) * s3 + o2 * (1.0 - s3)

    r3 = jax.nn.relu(o3)
    g4 = _gat(r3, src, dst, bp["g4"])
    s4 = jax.nn.sigmoid(adp(bp["l4"], r3))
    o4 = jax.nn.relu(g4) * s4 + o3 * (1.0 - s4)
    return o4


def kernel(x, edge_index, dtw_edge_index, params):
    seq = params["seq"]
    x1 = x @ seq["W"] + seq["b"] + x
    sp = _branch(x1, edge_index, params["sp"])
    x2 = x1 @ seq["W"] + seq["b"] + x1
    dtw = _branch(x2, dtw_edge_index, params["dtw"])
    return jnp.concatenate([sp, dtw], axis=1)


# speedup vs baseline: 1.6124x; 1.6124x over previous
import functools

import jax
import jax.numpy as jnp
from jax import lax
from jax.experimental import pallas as pl
from jax.experimental.pallas import tpu as pltpu
from jax.experimental.pallas import tpu_sc as plsc

N = 10000
E = 640000
SEQ = 12
GD = 16
NPAD = 10240
BM = 400
BK = 2048

NC, NS = 2, 16
NW = NC * NS
EC = 128
E2 = E + N
E2P = -(-E2 // (NW * EC * 2)) * (NW * EC * 2)
SPAN = E2P // NW
CH = SPAN // EC
NA = 10112
RPT = NA // NS


def _adp_body(q_ref, k_ref, v_ref, out_ref):
    q = q_ref[...]

    def body(i, acc):
        kc = k_ref[:, pl.ds(i * BK, BK)]
        logits = jnp.dot(q, kc, preferred_element_type=jnp.float32)
        p = jnp.exp(jnp.maximum(logits, 0.0))
        return acc + jnp.dot(p, v_ref[pl.ds(i * BK, BK), :],
                             preferred_element_type=jnp.float32)

    acc = jax.lax.fori_loop(
        0, NPAD // BK, body, jnp.zeros((BM, GD + 1), jnp.float32))
    out_ref[...] = acc[:, :GD] / acc[:, GD:GD + 1]


@jax.jit
def _adp_all(q, k_aug, v):
    ones = jnp.ones((N, 1), jnp.float32)
    v_aug = jnp.concatenate([v, ones], axis=1)
    v_aug = jnp.pad(v_aug, ((0, NPAD - N), (0, 0)))
    return pl.pallas_call(
        _adp_body,
        grid=(N // BM,),
        in_specs=[
            pl.BlockSpec((BM, q.shape[1]), lambda i: (i, 0)),
            pl.BlockSpec((q.shape[1], NPAD), lambda i: (0, 0)),
            pl.BlockSpec((NPAD, GD + 1), lambda i: (0, 0)),
        ],
        out_specs=pl.BlockSpec((BM, GD), lambda i: (i, 0)),
        out_shape=jax.ShapeDtypeStruct((N, GD), jnp.float32),
    )(q, k_aug, v_aug)


def _gat_body(heads, RW,
              h_hbm, as_hbm, ad_hbm, src_hbm, dst_hbm, out_hbm,
              as_v, ad_v, srcb, dstb, rows_ab, buf_ab, accum,
              sem_ra, sem_rb, sem_sa, sem_sb,
              sem_i0, sem_i1, sem_i2, sem_i3):
    c = lax.axis_index("c")
    s = lax.axis_index("s")
    wid = c * NS + s
    pltpu.sync_copy(as_hbm, as_v)
    pltpu.sync_copy(ad_hbm, ad_v)

    zeros16 = jnp.zeros((16,), jnp.float32)
    zoffs = [j * 16 for j in range(RW // 16)] + [RW - 16]

    def zrow(i, _):
        for ab in range(2):
            for off in zoffs:
                buf_ab[ab, i, pl.ds(off, 16)] = zeros16
        return 0

    lax.fori_loop(0, EC, zrow, 0)

    off, rem = 0, RPT
    while rem > 0:
        k = min(EC, rem)
        pltpu.sync_copy(buf_ab.at[0, pl.ds(0, k)],
                        accum.at[pl.ds(s * RPT + off, k)])
        off += k
        rem -= k
    plsc.subcore_barrier()

    lane = lax.iota(jnp.int32, 16)
    dencol = [jnp.full((16,), heads * GD + hd, jnp.int32)
              for hd in range(heads)]
    sem_r = [sem_ra, sem_rb]
    sem_s = [sem_sa, sem_sb]
    sem_i = [sem_i0, sem_i1, sem_i2, sem_i3]

    def issue_idx(g, sl):
        pltpu.async_copy(src_hbm.at[wid, g], srcb.at[sl], sem_i[sl])
        pltpu.async_copy(dst_hbm.at[wid, g], dstb.at[sl], sem_i[sl])

    def drain_idx(sl):
        pltpu.make_async_copy(src_hbm.at[0, 0], srcb.at[sl],
                              sem_i[sl]).wait()
        pltpu.make_async_copy(dst_hbm.at[0, 0], dstb.at[sl],
                              sem_i[sl]).wait()

    def issue_rows(sl, rs):
        for hd in range(heads):
            pltpu.async_copy(h_hbm.at[hd].at[srcb.at[sl]],
                             rows_ab.at[rs, hd], sem_r[rs])

    def drain_rows(rs):
        pltpu.make_async_copy(h_hbm.at[0].at[srcb.at[0]],
                              rows_ab.at[rs], sem_r[rs]).wait()

    def scatter(sl, rs):
        pltpu.async_copy(buf_ab.at[rs], accum.at[dstb.at[sl]], sem_s[rs],
                         add=True)

    def drain_scatter(rs):
        pltpu.make_async_copy(buf_ab.at[rs], accum.at[dstb.at[0]],
                              sem_s[rs]).wait()

    def compute(sl, rs):
        def grp(i, _):
            sv = srcb[sl, pl.ds(i * 16, 16)]
            dv = dstb[sl, pl.ds(i * 16, 16)]
            ev = i * 16 + lane
            for hd in range(heads):
                av = plsc.load_gather(as_v, [sv + hd * N])
                bv = plsc.load_gather(ad_v, [dv + hd * N])
                t = av + bv
                w = jnp.exp(jnp.where(t > 0, t, 0.2 * t))
                plsc.store_scatter(buf_ab.at[rs], [ev, dencol[hd]], w)
                for j in range(16):
                    e2 = i * 16 + j
                    buf_ab[rs, e2, pl.ds(hd * GD, GD)] = \
                        rows_ab[rs, hd, e2, :] * w[j]
            return 0

        lax.fori_loop(0, EC // 16, grp, 0)

    issue_idx(0, 0)
    issue_idx(1, 1)
    drain_idx(0)
    issue_rows(0, 0)

    def body(q, _):
        for c in range(4):
            g = 4 * q + c
            rs = c % 2
            if c < 2:
                @pl.when(q > 0)
                def _():
                    drain_scatter(rs)
            else:
                drain_scatter(rs)

            @pl.when(g + 2 < CH)
            def _():
                issue_idx(g + 2, (c + 2) % 4)

            @pl.when(g + 1 < CH)
            def _():
                drain_idx((c + 1) % 4)
                issue_rows((c + 1) % 4, (c + 1) % 2)

            drain_rows(rs)
            compute(c, rs)
            scatter(c, rs)
        return 0

    lax.fori_loop(0, CH // 4, body, 0)
    drain_scatter(0)
    drain_scatter(1)
    plsc.subcore_barrier()

    off, rem = 0, RPT
    while rem > 0:
        k = min(EC, rem)
        pltpu.sync_copy(accum.at[pl.ds(s * RPT + off, k)],
                        out_hbm.at[c].at[pl.ds(s * RPT + off, k)])
        off += k
        rem -= k


@functools.lru_cache(maxsize=None)
def _gat_call(heads):
    RW = -(-(heads * GD + heads) // 8) * 8
    mesh = plsc.VectorSubcoreMesh(core_axis_name="c", subcore_axis_name="s",
                                  num_cores=NC, num_subcores=NS)
    return pl.kernel(
        functools.partial(_gat_body, heads, RW),
        out_type=jax.ShapeDtypeStruct((NC, NA, RW), jnp.float32),
        mesh=mesh,
        compiler_params=pltpu.CompilerParams(needs_layout_passes=False,
                                             use_tc_tiling_on_sc=False),
        scratch_types=[
            pltpu.VMEM((heads * N,), jnp.float32),
            pltpu.VMEM((heads * N,), jnp.float32),
            pltpu.VMEM((4, EC), jnp.int32),
            pltpu.VMEM((4, EC), jnp.int32),
            pltpu.VMEM((2, heads, EC, GD), jnp.float32),
            pltpu.VMEM((2, EC, RW), jnp.float32),
            pltpu.VMEM_SHARED((NA, RW), jnp.float32),
        ] + [pltpu.SemaphoreType.DMA] * 8,
    )


def _gat(x, src, dst, p):
    heads = p["att_src"].shape[0]
    h = (x @ p["W"]).reshape(N, heads, GD)
    a_s = jnp.sum(h * p["att_src"][None, :, :], axis=-1)
    a_d = jnp.sum(h * p["att_dst"][None, :, :], axis=-1)
    hT = jnp.transpose(h, (1, 0, 2))
    partials = _gat_call(heads)(
        hT, a_s.T.reshape(-1), a_d.T.reshape(-1),
        src.reshape(NW, CH, EC), dst.reshape(NW, CH, EC))
    acc = partials[0] + partials[1]
    num = acc[:N, :heads * GD].reshape(N, heads, GD)
    den = acc[:N, heads * GD:heads * GD + heads]
    out = num / (den[:, :, None] + 1e-16)
    return jnp.mean(out, axis=1) + p["b"]


def _branch(x, ei, bp):
    loop = jnp.arange(N, dtype=jnp.int32)
    src = jnp.pad(jnp.concatenate([ei[0], loop]), (0, E2P - E2))
    dst = jnp.pad(jnp.concatenate([ei[1], loop]), (0, E2P - E2),
                  constant_values=N)
    q = bp["src_emb"]
    k_aug = jnp.pad(bp["tgt_emb"], ((0, 0), (0, NPAD - N)))

    def adp(lp, v):
        return _adp_all(q, k_aug, v @ lp["W"]) + lp["b"]

    g1 = _gat(x, src, dst, bp["g1"])
    s1 = jax.nn.sigmoid(adp(bp["l1"], x))
    origin = x @ bp["origin"]["W"] + bp["origin"]["b"]
    o1 = jnp.tanh(g1) * s1 + origin * (1.0 - s1)

    t1 = jnp.tanh(o1)
    g2 = _gat(t1, src, dst, bp["g2"])
    s2 = jax.nn.sigmoid(adp(bp["l2"], t1))
    o2 = jax.nn.leaky_relu(g2, 0.01) * s2 + o1 * (1.0 - s2)

    r2 = jax.nn.relu(o2)
    g3 = _gat(r2, src, dst, bp["g3"])
    s3 = jax.nn.sigmoid(adp(bp["l3"], r2))
    o3 = jax.nn.relu(g3) * s3 + o2 * (1.0 - s3)

    r3 = jax.nn.relu(o3)
    g4 = _gat(r3, src, dst, bp["g4"])
    s4 = jax.nn.sigmoid(adp(bp["l4"], r3))
    o4 = jax.nn.relu(g4) * s4 + o3 * (1.0 - s4)
    return o4


def kernel(x, edge_index, dtw_edge_index, params):
    seq = params["seq"]
    x1 = x @ seq["W"] + seq["b"] + x
    sp = _branch(x1, edge_index, params["sp"])
    x2 = x1 @ seq["W"] + seq["b"] + x1
    dtw = _branch(x2, dtw_edge_index, params["dtw"])
    return jnp.concatenate([sp, dtw], axis=1)
